# Initial kernel scaffold; baseline (speedup 1.0000x reference)
import jax
import jax.numpy as jnp
from jax.experimental import pallas as pl


def kernel(pred_boxes, pred_cls_conf, pred_position):
    raise NotImplementedError("write your pallas kernel here")


# speedup vs baseline: 6.7457x; 6.7457x over previous
import jax
import jax.numpy as jnp
from jax.experimental import pallas as pl
from jax.experimental.pallas import tpu as pltpu

H = 128
W = 128
C = 80
HW = H * W
K = 100
NEG = -1e30
BIG = 10**9


def _body(boxes_ref, cls_ref, out_ref, masked_ref, cm2_ref, val_s, hw_s, c_s):
    lane_c1 = jax.lax.broadcasted_iota(jnp.int32, (1, C), 1)
    lane_c2 = jax.lax.broadcasted_iota(jnp.int32, (W, C), 1)
    row_h = jax.lax.broadcasted_iota(jnp.int32, (H, 1), 0)
    row_w = jax.lax.broadcasted_iota(jnp.int32, (W, 1), 0)
    row_w2 = jax.lax.broadcasted_iota(jnp.int32, (W, C), 0)

    def pool_body(h, _):
        blk = cls_ref[0, pl.ds(h * W, W), :]

        def shift_w(x, d):
            pad = jnp.full((abs(d), C), NEG, jnp.float32)
            if d > 0:
                return jnp.concatenate([pad, x[:-d, :]], axis=0)
            return jnp.concatenate([x[-d:, :], pad], axis=0)

        m1 = blk
        for d in (-2, -1, 1, 2):
            m1 = jnp.maximum(m1, shift_w(blk, d))

        def shift_c(x, d):
            pad = jnp.full((W, abs(d)), NEG, jnp.float32)
            if d > 0:
                return jnp.concatenate([pad, x[:, :-d]], axis=1)
            return jnp.concatenate([x[:, -d:], pad], axis=1)

        hm = m1
        for d in (-2, -1, 1, 2):
            hm = jnp.maximum(hm, shift_c(m1, d))

        masked = jnp.where(blk == hm, blk, 0.0)
        masked_ref[pl.ds(h * W, W), :] = masked
        cm2_ref[pl.ds(h, 1), :] = jnp.max(masked, axis=0, keepdims=True)
        return 0

    jax.lax.fori_loop(0, H, pool_body, 0)

    def ext_body(i, _):
        cm2 = cm2_ref[:, :]
        m = jnp.max(cm2)
        eq = cm2 == m
        colmask = jnp.any(eq, axis=0, keepdims=True)
        cstar = jnp.min(jnp.where(colmask, lane_c1, BIG))
        eqc = eq & (lane_c2 == cstar)
        rowmask = jnp.any(eqc, axis=1, keepdims=True)
        hstar = jnp.min(jnp.where(rowmask, row_h, BIG))
        chunk = masked_ref[pl.ds(hstar * W, W), :]
        em = (chunk == m) & (lane_c2 == cstar)
        wmask = jnp.any(em, axis=1, keepdims=True)
        wstar = jnp.min(jnp.where(wmask, row_w, BIG))
        hw = hstar * W + wstar
        val_s[i] = m
        hw_s[i] = hw
        c_s[i] = cstar
        row = masked_ref[pl.ds(hw, 1), :]
        masked_ref[pl.ds(hw, 1), :] = jnp.where(lane_c1 == cstar, -1.0, row)
        upd = jnp.where((row_w2 == wstar) & (lane_c2 == cstar), -1.0, chunk)
        newcol = jnp.max(jnp.where(lane_c2 == cstar, upd, NEG))
        cm2_ref[:, :] = jnp.where((row_h == hstar) & (lane_c2[:H, :] == cstar),
                                  newcol, cm2)
        return 0

    jax.lax.fori_loop(0, K, ext_body, 0)

    def rest_body(k, _):
        hw = hw_s[k]
        row = masked_ref[pl.ds(hw, 1), :]
        masked_ref[pl.ds(hw, 1), :] = jnp.where(lane_c1 == c_s[k], val_s[k], row)
        return 0

    jax.lax.fori_loop(0, K, rest_body, 0)

    def gath_body(k, _):
        hw = hw_s[k]
        box = boxes_ref[0, pl.ds(hw, 1), :]
        clsrow = masked_ref[pl.ds(hw, 1), :]
        rowout = jnp.concatenate(
            [box * 4.0, jnp.ones((1, 1), jnp.float32), clsrow], axis=1)
        out_ref[0, pl.ds(k, 1), :] = rowout
        return 0

    jax.lax.fori_loop(0, K, gath_body, 0)


def kernel(pred_boxes, pred_cls_conf, pred_position):
    del pred_position
    B = pred_boxes.shape[0]
    boxes = pred_boxes.reshape(B, HW, 4)
    cls = pred_cls_conf.reshape(B, HW, C)
    return pl.pallas_call(
        _body,
        grid=(B,),
        in_specs=[pl.BlockSpec((1, HW, 4), lambda b: (b, 0, 0)),
                  pl.BlockSpec((1, HW, C), lambda b: (b, 0, 0))],
        out_specs=pl.BlockSpec((1, K, 85), lambda b: (b, 0, 0)),
        out_shape=jax.ShapeDtypeStruct((B, K, 85), jnp.float32),
        scratch_shapes=[pltpu.VMEM((HW, C), jnp.float32),
                        pltpu.VMEM((H, C), jnp.float32),
                        pltpu.SMEM((K,), jnp.float32),
                        pltpu.SMEM((K,), jnp.int32),
                        pltpu.SMEM((K,), jnp.int32)],
    )(boxes, cls)
